# baseline (speedup 1.0000x reference)
import functools

import jax
import jax.numpy as jnp
from jax import lax
from jax.experimental import pallas as pl
from jax.experimental.pallas import tpu as pltpu
from jax.experimental.pallas import tpu_sc as plsc

_D = 32
_CHUNK = 128
_GPB = 8
_BLK = _CHUNK * _GPB
_TL = 8192


@functools.cache
def _make_detile(V: int):
    lanes = _TL
    grid = (V + lanes - 1) // lanes

    def body(x_ref, o_ref):
        q = lanes // 4
        y = x_ref[...].T
        for j in range(4):
            o_ref[:, _D * j:_D * (j + 1)] = y[q * j:q * (j + 1), :]

    return pl.pallas_call(
        body,
        grid=(grid,),
        in_specs=[pl.BlockSpec((_D, lanes), lambda i: (0, i))],
        out_specs=pl.BlockSpec((lanes // 4, 128), lambda i: (i, 0)),
        out_shape=jax.ShapeDtypeStruct((grid * lanes // 4, 128), jnp.float32),
    )


@functools.cache
def _make_gather(N: int, B: int, H: int):
    info = plsc.get_sparse_core_info()
    nc, ns = info.num_cores, info.num_subcores
    nw = nc * ns
    n_w = N // nw
    n_blk = n_w // _BLK
    assert n_blk % 2 == 0 and n_blk >= 4
    bph = B // _BLK
    tpb = _BLK // 128
    mesh = plsc.VectorSubcoreMesh(core_axis_name="c", subcore_axis_name="s")

    @functools.partial(
        pl.kernel,
        mesh=mesh,
        out_type=jax.ShapeDtypeStruct((N * _D,), jnp.float32),
        scratch_types=[
            pltpu.VMEM((2, _GPB, _CHUNK), jnp.int32),
            pltpu.VMEM((2, _BLK, _D), jnp.float32),
            pltpu.VMEM((4, tpb * 1024), jnp.float32),
            pltpu.SemaphoreType.DMA,
            pltpu.SemaphoreType.DMA,
            pltpu.SemaphoreType.DMA,
        ],
        compiler_params=pltpu.CompilerParams(
            use_tc_tiling_on_sc=False, needs_layout_passes=False),
    )
    def k(idx_hbm, table_hbm, out_hbm, idx_v, rows_v, t_v, sg0, sg1, sw):
        sem_g = (sg0, sg1)
        wid = lax.axis_index("s") * nc + lax.axis_index("c")
        kb = wid * n_blk
        iota16 = lax.iota(jnp.int32, 16)

        def fire_block(b, p):
            irow = pl.multiple_of((kb + b) * (_BLK // _CHUNK), _GPB)
            pltpu.sync_copy(idx_hbm.at[pl.ds(irow, _GPB)], idx_v.at[p])
            for j in range(_GPB):
                pltpu.async_copy(
                    table_hbm.at[idx_v.at[p, j]],
                    rows_v.at[p, pl.ds(j * _CHUNK, _CHUNK)],
                    sem_g[p])

        def wait_gathers(p):
            pltpu.make_async_copy(
                table_hbm.at[pl.ds(0, _BLK)], rows_v.at[p], sem_g[p]).wait()

        def transpose(p):
            @plsc.parallel_loop(0, _BLK // 16, unroll=8)
            def tb(m):
                bti = m // 8
                c16 = m % 8
                row16 = iota16 + m * 16
                for dt in range(4):
                    for dr in range(8):
                        col = jnp.full((16,), dt * 8 + dr, jnp.int32)
                        v = plsc.load_gather(rows_v.at[p], [row16, col])
                        t_v[dt, pl.ds(bti * 1024 + dr * 128 + c16 * 16,
                                      16)] = v

        def fire_wb(b):
            g = kb + b
            h = g // bph
            bt0 = (g % bph) * tpb
            for dt in range(4):
                off = pl.multiple_of(
                    ((h * 4 + dt) * (B // 128) + bt0) * 1024, tpb * 1024)
                pltpu.async_copy(
                    t_v.at[dt], out_hbm.at[pl.ds(off, tpb * 1024)], sw)

        def wait_wb():
            for dt in range(4):
                pltpu.make_async_copy(
                    t_v.at[dt], out_hbm.at[pl.ds(0, tpb * 1024)], sw).wait()

        fire_block(0, 0)
        fire_block(1, 1)

        def step(i, carry):
            for p in (0, 1):
                b = 2 * i + p
                wait_gathers(p)

                @pl.when(b > 0)
                def _():
                    wait_wb()

                transpose(p)
                fire_wb(b)

                @pl.when(b + 2 < n_blk)
                def _():
                    fire_block(b + 2, p)
            return carry

        lax.fori_loop(0, n_blk // 2, step, 0)
        wait_wb()

    return k


def kernel(batch, table):
    B, H = batch.shape
    N = B * H
    V = table.shape[0]
    t_rm = _make_detile(V)(jnp.transpose(table))
    t_rm = t_rm.reshape(t_rm.shape[0] * 4, _D)
    qm = _TL // 4 - 1
    bt = jnp.transpose(batch).astype(jnp.int32)
    bt = (bt & -_TL) | ((bt & qm) << 2) | ((bt // (_TL // 4)) & 3)
    idx = bt.reshape(N // _CHUNK, _CHUNK)
    out1 = _make_gather(N, B, H)(idx, t_rm)
    out5 = out1.reshape(H, 4, B // 128, 8, 128)
    return jnp.transpose(out5, (2, 4, 0, 1, 3)).reshape(B, H, _D)


# speedup vs baseline: 2.5171x; 2.5171x over previous
import functools

import jax
import jax.numpy as jnp
from jax import lax
from jax.experimental import pallas as pl
from jax.experimental.pallas import tpu as pltpu
from jax.experimental.pallas import tpu_sc as plsc

_D = 32
_CHUNK = 128
_GPB = 8
_BLK = _CHUNK * _GPB
_C = 10
_TL = 8192


@functools.cache
def _make_detile(V: int):
    lanes = _TL
    grid = (V + lanes - 1) // lanes

    def body(x_ref, o_ref):
        q = lanes // 4
        y = x_ref[...].T
        for j in range(4):
            o_ref[:, _D * j:_D * (j + 1)] = y[q * j:q * (j + 1), :]

    return pl.pallas_call(
        body,
        grid=(grid,),
        in_specs=[pl.BlockSpec((_D, lanes), lambda i: (0, i))],
        out_specs=pl.BlockSpec((lanes // 4, 128), lambda i: (i, 0)),
        out_shape=jax.ShapeDtypeStruct((grid * lanes // 4, 128), jnp.float32),
    )


@functools.cache
def _make_gather(N: int, B: int, c: int):
    info = plsc.get_sparse_core_info()
    nc, ns = info.num_cores, info.num_subcores
    nw = nc * ns
    n_ch = N // _C
    n_w = n_ch // nw
    n_blk = n_w // _BLK
    assert n_blk % 2 == 0 and n_blk >= 4
    bph = B // _BLK
    q = B // 4
    h0 = c * (n_ch // B)
    blk0 = c * (n_ch // _BLK)
    mesh = plsc.VectorSubcoreMesh(core_axis_name="c", subcore_axis_name="s")

    @functools.partial(
        pl.kernel,
        mesh=mesh,
        out_type=jax.ShapeDtypeStruct((n_ch // 4, 128), jnp.float32),
        scratch_types=[
            pltpu.VMEM((2, _GPB, _CHUNK), jnp.int32),
            pltpu.VMEM((2, _BLK, _D), jnp.float32),
            pltpu.SemaphoreType.DMA,
            pltpu.SemaphoreType.DMA,
            pltpu.SemaphoreType.DMA,
            pltpu.SemaphoreType.DMA,
        ],
        compiler_params=pltpu.CompilerParams(use_tc_tiling_on_sc=False),
    )
    def k(idx_hbm, table_hbm, out_hbm, idx_v, rows_v, sg0, sg1, sw0, sw1):
        sem_g = (sg0, sg1)
        sem_w = (sw0, sw1)
        wid = lax.axis_index("s") * nc + lax.axis_index("c")
        kb = blk0 + wid * n_blk

        def fire_block(b, p):
            irow = pl.multiple_of((kb + b) * (_BLK // _CHUNK), _GPB)
            pltpu.sync_copy(idx_hbm.at[pl.ds(irow, _GPB)], idx_v.at[p])
            for j in range(_GPB):
                pltpu.async_copy(
                    table_hbm.at[idx_v.at[p, j]],
                    rows_v.at[p, pl.ds(j * _CHUNK, _CHUNK)],
                    sem_g[p])

        def wait_gathers(p):
            pltpu.make_async_copy(
                table_hbm.at[pl.ds(0, _BLK)], rows_v.at[p], sem_g[p]).wait()

        def fire_wb(b, p):
            g = kb + b
            h = g // bph
            b0 = (g % bph) * _BLK
            j0 = b0 // q
            row = pl.multiple_of((h - h0) * (B // 4) + (b0 - j0 * q), _BLK)
            pltpu.async_copy(
                rows_v.at[p],
                out_hbm.at[pl.ds(row, _BLK), pl.ds(j0 * _D, _D)],
                sem_w[p])

        def wait_wb(p):
            pltpu.make_async_copy(
                rows_v.at[p],
                out_hbm.at[pl.ds(0, _BLK), pl.ds(0, _D)],
                sem_w[p]).wait()

        fire_block(0, 0)
        fire_block(1, 1)
        wait_gathers(0)
        fire_wb(0, 0)

        def step(i, carry):
            g = 2 * i
            for p in (0, 1):
                b = g + p
                wait_wb(p)
                fire_block(b, p)
                wait_gathers(1 - p)
                fire_wb(b - 1, 1 - p)
            return carry

        lax.fori_loop(1, n_blk // 2, step, 0)

        wait_gathers(1)
        fire_wb(n_blk - 1, 1)
        wait_wb(0)
        wait_wb(1)

    return k


@functools.cache
def _make_transpose(B: int, H: int, c: int):
    nb = B // 4
    Hc = H // _C
    h0 = c * Hc

    def body(x_ref, *rest):
        o_ref = rest[-1]
        xt = x_ref[0].T
        for j in range(4):
            o_ref[0, :, j * nb:(j + 1) * nb] = xt[_D * j:_D * (j + 1), :]

    in_specs = [pl.BlockSpec((1, nb, 128), lambda h: (h, 0, 0))]
    aliases = {}
    if c > 0:
        in_specs.append(pl.BlockSpec(memory_space=pltpu.MemorySpace.HBM))
        aliases = {1: 0}

    return pl.pallas_call(
        body,
        grid=(Hc,),
        in_specs=in_specs,
        out_specs=pl.BlockSpec((1, _D, B), lambda h: (h + h0, 0, 0)),
        out_shape=jax.ShapeDtypeStruct((H, _D, B), jnp.float32),
        input_output_aliases=aliases,
    )


def kernel(batch, table):
    B, H = batch.shape
    N = B * H
    V = table.shape[0]
    t_rm = _make_detile(V)(jnp.transpose(table))
    t_rm = t_rm.reshape(t_rm.shape[0] * 4, _D)
    qm = _TL // 4 - 1
    bt = jnp.transpose(batch).astype(jnp.int32)
    bt = (bt & -_TL) | ((bt & qm) << 2) | ((bt // (_TL // 4)) & 3)
    idx = bt.reshape(N // _CHUNK, _CHUNK)
    gs = [_make_gather(N, B, c)(idx, t_rm) for c in range(_C)]
    out_t = None
    for c in range(_C):
        g3 = gs[c].reshape(H // _C, B // 4, 128)
        args = (g3,) if c == 0 else (g3, out_t)
        out_t = _make_transpose(B, H, c)(*args)
    return jnp.transpose(out_t, (2, 0, 1))
